# baseline (speedup 1.0000x reference)
import jax
import jax.numpy as jnp
from jax.experimental import pallas as pl
from jax.experimental.pallas import tpu as pltpu

H, W, DH = 32, 32, 384
NCHUNK = 8
CH = H // NCHUNK


def _body(rowv, colv, out_hbm, buf, sems):
    colb = jnp.broadcast_to(colv[...][None, :, :], (CH, W, DH))
    for k in range(NCHUNK):
        buf[CH * k:CH * (k + 1), :, 0:DH] = colb
    copies = []
    for k in range(NCHUNK):
        row = rowv[CH * k:CH * (k + 1), :]
        buf[CH * k:CH * (k + 1), :, DH:2 * DH] = jnp.broadcast_to(
            row[:, None, :], (CH, W, DH))
        cp = pltpu.make_async_copy(
            buf.at[pl.ds(CH * k, CH)],
            out_hbm.at[pl.ds(CH * k, CH)],
            sems.at[k],
        )
        cp.start()
        copies.append(cp)
    for cp in copies:
        cp.wait()


def kernel(h, w, row_embed, col_embed):
    return pl.pallas_call(
        _body,
        grid=(1,),
        in_specs=[
            pl.BlockSpec((H, DH), lambda i: (0, 0)),
            pl.BlockSpec((W, DH), lambda i: (0, 0)),
        ],
        out_specs=pl.BlockSpec(memory_space=pl.ANY, index_map=lambda i: (0, 0, 0)),
        out_shape=jax.ShapeDtypeStruct((H, W, 2 * DH), jnp.float32),
        scratch_shapes=[
            pltpu.VMEM((H, W, 2 * DH), jnp.float32),
            pltpu.SemaphoreType.DMA((NCHUNK,)),
        ],
    )(row_embed, col_embed)
